# Initial kernel scaffold; baseline (speedup 1.0000x reference)
import jax
import jax.numpy as jnp
from jax.experimental import pallas as pl


def kernel(xyz, new_xyz):
    raise NotImplementedError("write your pallas kernel here")


# speedup vs baseline: 8.6396x; 8.6396x over previous
import functools

import jax
import jax.numpy as jnp
from jax.experimental import pallas as pl

K = 32
Q_BLK = 128


def _knn_kernel(q_ref, pt_ref, o_ref):
    q = q_ref[0]
    pt = pt_ref[0]
    n = pt.shape[1]
    qn = jnp.sum(q * q, axis=1, keepdims=True)
    pn = jnp.sum(pt * pt, axis=0, keepdims=True)
    d = jax.lax.dot_general(
        q, pt, (((1,), (0,)), ((), ())),
        preferred_element_type=jnp.float32)
    d = (-2.0 * d + qn) + pn
    col = jax.lax.broadcasted_iota(jnp.int32, d.shape, 1)
    big = jnp.float32(jnp.inf)
    for k in range(K):
        m = jnp.min(d, axis=1, keepdims=True)
        amin = jnp.min(jnp.where(d == m, col, jnp.int32(n)), axis=1)
        o_ref[0, :, k] = amin
        d = jnp.where(col == amin[:, None], big, d)


def kernel(xyz, new_xyz):
    b, n, _ = xyz.shape
    m = new_xyz.shape[1]
    xyz_t = jnp.swapaxes(xyz, 1, 2)
    grid = (b, m // Q_BLK)
    return pl.pallas_call(
        _knn_kernel,
        grid=grid,
        in_specs=[
            pl.BlockSpec((1, Q_BLK, 3), lambda bi, qi: (bi, qi, 0)),
            pl.BlockSpec((1, 3, n), lambda bi, qi: (bi, 0, 0)),
        ],
        out_specs=pl.BlockSpec((1, Q_BLK, K), lambda bi, qi: (bi, qi, 0)),
        out_shape=jax.ShapeDtypeStruct((b, m, K), jnp.int32),
    )(new_xyz, xyz_t)
